# baseline (speedup 1.0000x reference)
import functools

import jax
import jax.numpy as jnp
from jax import lax
from jax.experimental import pallas as pl
from jax.experimental.pallas import tpu as pltpu
from jax.experimental.pallas import tpu_sc as plsc

QB = 2
NBUF = 2
LP = 56


@functools.cache
def _make_gather(Bq, L, V, D):
    info = plsc.get_sparse_core_info()
    NC, NS = info.num_cores, info.num_subcores
    NW = NC * NS
    assert Bq % (NW * QB) == 0
    q_per_w = Bq // NW
    n_chunks = q_per_w // QB
    mesh = plsc.VectorSubcoreMesh(core_axis_name="c", subcore_axis_name="s")

    @functools.partial(
        pl.kernel,
        mesh=mesh,
        out_type=jax.ShapeDtypeStruct((Bq, L, D), jnp.float32),
        scratch_types=[
            pltpu.VMEM((q_per_w * LP,), jnp.int32),
            pltpu.VMEM((NBUF, QB, LP, 2 * D), jnp.float32),
            pltpu.VMEM((NBUF, QB, L, D), jnp.float32),
            pltpu.SemaphoreType.DMA((NBUF,)),
        ],
    )
    def k(table_hbm, idx_hbm, out_hbm, idx_v, g_v, o_v, gsem):
        wid = lax.axis_index("s") * NC + lax.axis_index("c")
        q0 = wid * q_per_w
        pltpu.sync_copy(idx_hbm.at[pl.ds(q0 * LP, q_per_w * LP)], idx_v)

        def start(i, b):
            for j in range(QB):
                pltpu.async_copy(
                    table_hbm.at[idx_v.at[pl.ds((i * QB + j) * LP, LP)]],
                    g_v.at[b].at[j],
                    gsem.at[b],
                )

        def drain(i, b):
            for j in range(QB):
                pltpu.make_async_copy(
                    table_hbm.at[pl.ds(0, LP)],
                    g_v.at[b].at[j],
                    gsem.at[b],
                ).wait()

        def compact(i, b):
            for j in range(QB):
                for l in range(L):
                    for c in range(D // 16):
                        o_v[b, j, l, pl.ds(c * 16, 16)] = g_v[
                            b, j, l, pl.ds(c * 16, 16)
                        ]

        def store(i, b):
            pltpu.sync_copy(o_v.at[b], out_hbm.at[pl.ds(q0 + i * QB, QB)])

        for b in range(NBUF):
            start(b, b)

        def body(i, _):
            b = i % NBUF
            drain(i, b)
            compact(i, b)
            store(i, b)
            start(i + NBUF, b)
            return ()

        lax.fori_loop(0, n_chunks - NBUF, body, ())

        for i in range(n_chunks - NBUF, n_chunks):
            b = i % NBUF
            drain(i, b)
            compact(i, b)
            store(i, b)

    return k


def kernel(indices, table):
    Bq, L = indices.shape
    V, D = table.shape
    table_p = jnp.pad(table, ((0, 0), (0, 2 * D - D)))
    idx_pf = jnp.pad(indices, ((0, 0), (0, LP - L))).reshape(Bq * LP)
    return _make_gather(Bq, L, V, D)(table_p, idx_pf)


# speedup vs baseline: 4.3512x; 4.3512x over previous
import functools

import jax
import jax.numpy as jnp
from jax import lax
from jax.experimental import pallas as pl
from jax.experimental.pallas import tpu as pltpu
from jax.experimental.pallas import tpu_sc as plsc

QB = 8
NBUF = 2


@functools.cache
def _make_gather(Bq, L, V, D):
    info = plsc.get_sparse_core_info()
    NC, NS = info.num_cores, info.num_subcores
    NW = NC * NS
    assert Bq % (NW * QB) == 0
    q_per_w = Bq // NW
    n_chunks = q_per_w // QB
    mesh = plsc.VectorSubcoreMesh(core_axis_name="c", subcore_axis_name="s")

    @functools.partial(
        pl.kernel,
        mesh=mesh,
        out_type=jax.ShapeDtypeStruct((Bq, L, D), jnp.float32),
        compiler_params=pltpu.CompilerParams(use_tc_tiling_on_sc=False),
        scratch_types=[
            pltpu.VMEM((q_per_w, L), jnp.int32),
            pltpu.VMEM((NBUF, QB, L, D), jnp.float32),
            pltpu.SemaphoreType.DMA((NBUF,)),
        ],
    )
    def k(table_hbm, idx_hbm, out_hbm, idx_v, rows_v, gsem):
        wid = lax.axis_index("s") * NC + lax.axis_index("c")
        q0 = wid * q_per_w
        pltpu.sync_copy(idx_hbm.at[pl.ds(q0, q_per_w)], idx_v)

        def start(i, b):
            for j in range(QB):
                pltpu.async_copy(
                    table_hbm.at[idx_v.at[i * QB + j]],
                    rows_v.at[b].at[j],
                    gsem.at[b],
                )

        def drain(i, b):
            pltpu.make_async_copy(
                out_hbm.at[pl.ds(q0 + i * QB, QB)],
                rows_v.at[b],
                gsem.at[b],
            ).wait()

        def store(i, b):
            pltpu.sync_copy(rows_v.at[b], out_hbm.at[pl.ds(q0 + i * QB, QB)])

        for b in range(NBUF):
            start(b, b)

        def body(i, _):
            b = i % NBUF
            drain(i, b)
            store(i, b)
            start(i + NBUF, b)
            return ()

        lax.fori_loop(0, n_chunks - NBUF, body, ())

        for i in range(n_chunks - NBUF, n_chunks):
            b = i % NBUF
            drain(i, b)
            store(i, b)

    return k


def kernel(indices, table):
    Bq, L = indices.shape
    V, D = table.shape
    return _make_gather(Bq, L, V, D)(table, indices)
